# Initial kernel scaffold; baseline (speedup 1.0000x reference)
import jax
import jax.numpy as jnp
from jax.experimental import pallas as pl


def kernel(hidden_states, router_logits, w_gate, w_up, w_down, per_expert_scale):
    raise NotImplementedError("write your pallas kernel here")


# speedup vs baseline: 1.8126x; 1.8126x over previous
import jax
import jax.numpy as jnp
from jax.experimental import pallas as pl
from jax.experimental.pallas import tpu as pltpu

T = 64
D = 768
E = 64
F = 1024


def _moe_body(h_ref, logits_ref, scale_ref, wg_ref, wu_ref, wd_ref,
              out_ref, gates_ref):
    e = pl.program_id(0)

    @pl.when(e == 0)
    def _():
        logits = logits_ref[...]
        lane = jax.lax.broadcasted_iota(jnp.int32, (T, E), 1)
        a1 = jnp.argmax(logits, axis=1)
        oh1 = lane == a1[:, None]
        masked = jnp.where(oh1, -jnp.inf, logits)
        a2 = jnp.argmax(masked, axis=1)
        oh2 = lane == a2[:, None]
        probs = jax.nn.softmax(logits, axis=1)
        sel = jnp.where(oh1 | oh2, probs, 0.0)
        renorm = jnp.sum(sel, axis=1, keepdims=True)
        renorm = jnp.where(renorm > 0.0, renorm, 1.0)
        gates_ref[...] = sel / renorm * scale_ref[...]
        out_ref[...] = jnp.zeros_like(out_ref)

    h = h_ref[...]
    g = jax.nn.gelu(jnp.dot(h, wg_ref[0], preferred_element_type=jnp.float32))
    u = jnp.dot(h, wu_ref[0], preferred_element_type=jnp.float32)
    y = jnp.dot(g * u, wd_ref[0], preferred_element_type=jnp.float32)
    lane = jax.lax.broadcasted_iota(jnp.int32, (T, E), 1)
    gcol = jnp.sum(jnp.where(lane == e, gates_ref[...], 0.0),
                   axis=1, keepdims=True)
    out_ref[...] += y * gcol


def kernel(hidden_states, router_logits, w_gate, w_up, w_down,
           per_expert_scale):
    scale2d = per_expert_scale.reshape(1, E)
    return pl.pallas_call(
        _moe_body,
        grid=(E,),
        in_specs=[
            pl.BlockSpec((T, D), lambda e: (0, 0)),
            pl.BlockSpec((T, E), lambda e: (0, 0)),
            pl.BlockSpec((1, E), lambda e: (0, 0)),
            pl.BlockSpec((1, D, F), lambda e: (e, 0, 0)),
            pl.BlockSpec((1, D, F), lambda e: (e, 0, 0)),
            pl.BlockSpec((1, F, D), lambda e: (e, 0, 0)),
        ],
        out_specs=pl.BlockSpec((T, D), lambda e: (0, 0)),
        out_shape=jax.ShapeDtypeStruct((T, D), jnp.float32),
        scratch_shapes=[pltpu.VMEM((T, E), jnp.float32)],
        compiler_params=pltpu.CompilerParams(
            dimension_semantics=("arbitrary",)),
    )(hidden_states, router_logits, scale2d, w_gate, w_up, w_down)


# speedup vs baseline: 1.8426x; 1.0166x over previous
import jax
import jax.numpy as jnp
from jax.experimental import pallas as pl
from jax.experimental.pallas import tpu as pltpu

T = 64
D = 768
E = 64
F = 1024


def _route_body(logits_ref, scale_ref, gates_ref, perm_ref):
    logits = logits_ref[...]
    lane = jax.lax.broadcasted_iota(jnp.int32, (T, E), 1)
    a1 = jnp.argmax(logits, axis=1)
    oh1 = lane == a1[:, None]
    masked = jnp.where(oh1, -jnp.inf, logits)
    a2 = jnp.argmax(masked, axis=1)
    oh2 = lane == a2[:, None]
    probs = jax.nn.softmax(logits, axis=1)
    sel = jnp.where(oh1 | oh2, probs, 0.0)
    renorm = jnp.sum(sel, axis=1, keepdims=True)
    renorm = jnp.where(renorm > 0.0, renorm, 1.0)
    gates_ref[...] = sel / renorm * scale_ref[...]

    cnt = jnp.sum((oh1 | oh2).astype(jnp.int32), axis=0)
    active = cnt > 0
    rowi = jax.lax.broadcasted_iota(jnp.int32, (E, E), 0)
    coli = jax.lax.broadcasted_iota(jnp.int32, (E, E), 1)
    prev = jnp.max(jnp.where((coli <= rowi) & active[None, :], coli, -1),
                   axis=1)
    iota_e = jax.lax.iota(jnp.int32, E)
    first_active = jnp.min(jnp.where(active, iota_e, E - 1))
    perm_ref[...] = jnp.where(prev < 0, first_active, prev).reshape(1, E)


def _moe_body(perm_ref, h_ref, gates_ref, wg_ref, wu_ref, wd_ref, out_ref):
    e = pl.program_id(0)

    @pl.when(e == 0)
    def _():
        out_ref[...] = jnp.zeros_like(out_ref)

    @pl.when(perm_ref[e] == e)
    def _():
        h = h_ref[...]
        g = jax.nn.gelu(
            jnp.dot(h, wg_ref[0], preferred_element_type=jnp.float32))
        u = jnp.dot(h, wu_ref[0], preferred_element_type=jnp.float32)
        y = jnp.dot(g * u, wd_ref[0], preferred_element_type=jnp.float32)
        lane = jax.lax.broadcasted_iota(jnp.int32, (T, E), 1)
        gcol = jnp.sum(jnp.where(lane == e, gates_ref[...], 0.0),
                       axis=1, keepdims=True)
        out_ref[...] += y * gcol


def kernel(hidden_states, router_logits, w_gate, w_up, w_down,
           per_expert_scale):
    scale2d = per_expert_scale.reshape(1, E)
    gates, perm2d = pl.pallas_call(
        _route_body,
        in_specs=[
            pl.BlockSpec((T, E), lambda: (0, 0)),
            pl.BlockSpec((1, E), lambda: (0, 0)),
        ],
        out_specs=[
            pl.BlockSpec((T, E), lambda: (0, 0)),
            pl.BlockSpec((1, E), lambda: (0, 0)),
        ],
        out_shape=[
            jax.ShapeDtypeStruct((T, E), jnp.float32),
            jax.ShapeDtypeStruct((1, E), jnp.int32),
        ],
    )(router_logits, scale2d)
    perm = perm2d.reshape(E)

    return pl.pallas_call(
        _moe_body,
        grid_spec=pltpu.PrefetchScalarGridSpec(
            num_scalar_prefetch=1,
            grid=(E,),
            in_specs=[
                pl.BlockSpec((T, D), lambda e, p: (0, 0)),
                pl.BlockSpec((T, E), lambda e, p: (0, 0)),
                pl.BlockSpec((1, D, F), lambda e, p: (p[e], 0, 0)),
                pl.BlockSpec((1, D, F), lambda e, p: (p[e], 0, 0)),
                pl.BlockSpec((1, F, D), lambda e, p: (p[e], 0, 0)),
            ],
            out_specs=pl.BlockSpec((T, D), lambda e, p: (0, 0)),
        ),
        out_shape=jax.ShapeDtypeStruct((T, D), jnp.float32),
        compiler_params=pltpu.CompilerParams(
            dimension_semantics=("arbitrary",)),
    )(perm, hidden_states, gates, w_gate, w_up, w_down)


# speedup vs baseline: 2.0454x; 1.1100x over previous
import jax
import jax.numpy as jnp
from jax.experimental import pallas as pl
from jax.experimental.pallas import tpu as pltpu

T = 64
D = 768
E = 64
F = 1024


def _route_body(logits_ref, scale_ref, gates_ref, alist_ref, cnt_ref):
    logits = logits_ref[...]
    lane = jax.lax.broadcasted_iota(jnp.int32, (T, E), 1)
    a1 = jnp.argmax(logits, axis=1)
    oh1 = lane == a1[:, None]
    masked = jnp.where(oh1, -jnp.inf, logits)
    a2 = jnp.argmax(masked, axis=1)
    oh2 = lane == a2[:, None]
    probs = jax.nn.softmax(logits, axis=1)
    sel = jnp.where(oh1 | oh2, probs, 0.0)
    renorm = jnp.sum(sel, axis=1, keepdims=True)
    renorm = jnp.where(renorm > 0.0, renorm, 1.0)
    gates_ref[...] = sel / renorm * scale_ref[...]

    cnt = jnp.sum((oh1 | oh2).astype(jnp.int32), axis=0)
    active = cnt > 0
    rowi = jax.lax.broadcasted_iota(jnp.int32, (E, E), 0)
    coli = jax.lax.broadcasted_iota(jnp.int32, (E, E), 1)
    before = (coli < rowi) & active[None, :]
    rank = jnp.sum(before.astype(jnp.int32), axis=1)
    hits = active[None, :] & (rank[None, :] == rowi)
    alist = jnp.sum(jnp.where(hits, coli, 0), axis=1)
    alist_ref[...] = alist.reshape(1, E)
    cnt_ref[...] = jnp.sum(active.astype(jnp.int32)).reshape(1, 1)


def _moe_body(alist_ref, cnt_ref, h_ref, gates_ref, wg_hbm, wu_hbm, wd_hbm,
              out_ref, wg_buf, wu_buf, wd_buf, sems):
    i = pl.program_id(0)
    n = cnt_ref[0]

    def start(j, slot):
        eid = alist_ref[j]
        pltpu.make_async_copy(wg_hbm.at[eid], wg_buf.at[slot],
                              sems.at[slot, 0]).start()
        pltpu.make_async_copy(wu_hbm.at[eid], wu_buf.at[slot],
                              sems.at[slot, 1]).start()
        pltpu.make_async_copy(wd_hbm.at[eid], wd_buf.at[slot],
                              sems.at[slot, 2]).start()

    @pl.when(i == 0)
    def _():
        out_ref[...] = jnp.zeros_like(out_ref)
        start(0, 0)

    @pl.when(i + 1 < n)
    def _():
        start(i + 1, jax.lax.rem(i + 1, 2))

    @pl.when(i < n)
    def _():
        slot = jax.lax.rem(i, 2)
        eid = alist_ref[i]
        pltpu.make_async_copy(wg_hbm.at[eid], wg_buf.at[slot],
                              sems.at[slot, 0]).wait()
        pltpu.make_async_copy(wu_hbm.at[eid], wu_buf.at[slot],
                              sems.at[slot, 1]).wait()
        pltpu.make_async_copy(wd_hbm.at[eid], wd_buf.at[slot],
                              sems.at[slot, 2]).wait()
        h = h_ref[...]
        g = jax.nn.gelu(
            jnp.dot(h, wg_buf[slot], preferred_element_type=jnp.float32))
        u = jnp.dot(h, wu_buf[slot], preferred_element_type=jnp.float32)
        y = jnp.dot(g * u, wd_buf[slot], preferred_element_type=jnp.float32)
        lane = jax.lax.broadcasted_iota(jnp.int32, (T, E), 1)
        gcol = jnp.sum(jnp.where(lane == eid, gates_ref[...], 0.0),
                       axis=1, keepdims=True)
        out_ref[...] += y * gcol


def kernel(hidden_states, router_logits, w_gate, w_up, w_down,
           per_expert_scale):
    scale2d = per_expert_scale.reshape(1, E)
    gates, alist2d, cnt2d = pl.pallas_call(
        _route_body,
        in_specs=[
            pl.BlockSpec((T, E), lambda: (0, 0)),
            pl.BlockSpec((1, E), lambda: (0, 0)),
        ],
        out_specs=[
            pl.BlockSpec((T, E), lambda: (0, 0)),
            pl.BlockSpec((1, E), lambda: (0, 0)),
            pl.BlockSpec((1, 1), lambda: (0, 0)),
        ],
        out_shape=[
            jax.ShapeDtypeStruct((T, E), jnp.float32),
            jax.ShapeDtypeStruct((1, E), jnp.int32),
            jax.ShapeDtypeStruct((1, 1), jnp.int32),
        ],
    )(router_logits, scale2d)

    return pl.pallas_call(
        _moe_body,
        grid_spec=pltpu.PrefetchScalarGridSpec(
            num_scalar_prefetch=2,
            grid=(E,),
            in_specs=[
                pl.BlockSpec((T, D), lambda i, a, c: (0, 0)),
                pl.BlockSpec((T, E), lambda i, a, c: (0, 0)),
                pl.BlockSpec(memory_space=pl.ANY),
                pl.BlockSpec(memory_space=pl.ANY),
                pl.BlockSpec(memory_space=pl.ANY),
            ],
            out_specs=pl.BlockSpec((T, D), lambda i, a, c: (0, 0)),
            scratch_shapes=[
                pltpu.VMEM((2, D, F), jnp.float32),
                pltpu.VMEM((2, D, F), jnp.float32),
                pltpu.VMEM((2, F, D), jnp.float32),
                pltpu.SemaphoreType.DMA((2, 3)),
            ],
        ),
        out_shape=jax.ShapeDtypeStruct((T, D), jnp.float32),
        compiler_params=pltpu.CompilerParams(
            dimension_semantics=("arbitrary",)),
    )(alist2d.reshape(E), cnt2d.reshape(1), hidden_states, gates,
      w_gate, w_up, w_down)
